# Initial kernel scaffold; baseline (speedup 1.0000x reference)
import jax
import jax.numpy as jnp
from jax.experimental import pallas as pl


def kernel(input_ids, position_ids, token_w, position_w):
    raise NotImplementedError("write your pallas kernel here")


# speedup vs baseline: 1.9058x; 1.9058x over previous
import functools

import jax
import jax.numpy as jnp
from jax import lax
from jax.experimental import pallas as pl
from jax.experimental.pallas import tpu as pltpu
from jax.experimental.pallas import tpu_sc as plsc

VOCAB = 49408
MAX_POS = 77
D = 768
SEQ = 77

NB = 8
NWORK = 10
PAD = NB * NWORK
LANES = 16
NCHUNK = D // LANES


def _make_kernel():
    info = plsc.get_sparse_core_info()
    nc = info.num_cores

    mesh = plsc.VectorSubcoreMesh(core_axis_name="c", subcore_axis_name="s")

    @functools.partial(
        pl.kernel,
        mesh=mesh,
        out_type=jax.ShapeDtypeStruct((PAD, D), jnp.float32),
        scratch_types=[
            pltpu.VMEM((NB,), jnp.int32),
            pltpu.VMEM((NB,), jnp.int32),
            pltpu.VMEM((NB, D), jnp.float32),
            pltpu.VMEM((NB, D), jnp.float32),
            pltpu.SemaphoreType.DMA,
            pltpu.SemaphoreType.DMA,
        ],
    )
    def emb_kernel(ids_hbm, pids_hbm, tok_hbm, posw_hbm, out_hbm,
                   idx_v, pidx_v, tok_v, pos_v, sem_t, sem_p):
        wid = lax.axis_index("s") * nc + lax.axis_index("c")

        @pl.when(wid < NWORK)
        def _():
            base = wid * NB
            pltpu.sync_copy(ids_hbm.at[pl.ds(base, NB)], idx_v)
            pltpu.sync_copy(pids_hbm.at[pl.ds(base, NB)], pidx_v)
            cp_t = pltpu.async_copy(tok_hbm.at[idx_v], tok_v, sem_t)
            cp_p = pltpu.async_copy(posw_hbm.at[pidx_v], pos_v, sem_p)
            cp_t.wait()
            cp_p.wait()
            for i in range(NB):
                for j in range(NCHUNK):
                    sl = pl.ds(j * LANES, LANES)
                    tok_v[i, sl] = tok_v[i, sl] + pos_v[i, sl]
            pltpu.sync_copy(tok_v, out_hbm.at[pl.ds(base, NB)])

    return emb_kernel


_emb_kernel = _make_kernel()


def kernel(input_ids, position_ids, token_w, position_w):
    ids = input_ids.astype(jnp.int32)
    pids = position_ids.astype(jnp.int32)
    pad = PAD - SEQ
    ids_p = jnp.concatenate([ids, jnp.zeros((pad,), jnp.int32)])
    pids_p = jnp.concatenate([pids, jnp.zeros((pad,), jnp.int32)])
    out = _emb_kernel(ids_p, pids_p, token_w, position_w)
    return out[None, :SEQ, :]


# speedup vs baseline: 2.1170x; 1.1108x over previous
import functools

import jax
import jax.numpy as jnp
from jax import lax
from jax.experimental import pallas as pl
from jax.experimental.pallas import tpu as pltpu
from jax.experimental.pallas import tpu_sc as plsc

VOCAB = 49408
MAX_POS = 77
D = 768
SEQ = 77

NB = 8
NWORK = 10
PAD = NB * NWORK
LANES = 16
NCHUNK = D // LANES


def _make_kernel():
    info = plsc.get_sparse_core_info()
    nc = info.num_cores

    mesh = plsc.VectorSubcoreMesh(core_axis_name="c", subcore_axis_name="s")

    @functools.partial(
        pl.kernel,
        mesh=mesh,
        out_type=jax.ShapeDtypeStruct((PAD, D), jnp.float32),
        scratch_types=[
            pltpu.VMEM((NB,), jnp.int32),
            pltpu.VMEM((NB,), jnp.int32),
            pltpu.VMEM((NB, D), jnp.float32),
            pltpu.VMEM((NB, D), jnp.float32),
            pltpu.SemaphoreType.DMA,
            pltpu.SemaphoreType.DMA,
            pltpu.SemaphoreType.DMA,
            pltpu.SemaphoreType.DMA,
        ],
    )
    def emb_kernel(ids_hbm, pids_hbm, tok_hbm, posw_hbm, out_hbm,
                   idx_v, pidx_v, tok_v, pos_v, sem_i, sem_pi, sem_t, sem_p):
        wid = lax.axis_index("s") * nc + lax.axis_index("c")

        @pl.when(wid < NWORK)
        def _():
            base = wid * NB
            cp_i = pltpu.async_copy(ids_hbm.at[pl.ds(base, NB)], idx_v, sem_i)
            cp_pi = pltpu.async_copy(pids_hbm.at[pl.ds(base, NB)], pidx_v, sem_pi)
            cp_i.wait()
            cp_t = pltpu.async_copy(tok_hbm.at[idx_v], tok_v, sem_t)
            cp_pi.wait()
            cp_p = pltpu.async_copy(posw_hbm.at[pidx_v], pos_v, sem_p)
            cp_t.wait()
            cp_p.wait()

            def add_body(j, carry):
                sl = pl.ds(j * LANES, LANES)
                for i in range(NB):
                    tok_v[i, sl] = tok_v[i, sl] + pos_v[i, sl]
                return carry

            lax.fori_loop(0, NCHUNK, add_body, 0)
            pltpu.sync_copy(tok_v, out_hbm.at[pl.ds(base, NB)])

    return emb_kernel


_emb_kernel = _make_kernel()


def kernel(input_ids, position_ids, token_w, position_w):
    ids = input_ids.astype(jnp.int32)
    pids = position_ids.astype(jnp.int32)
    pad = PAD - SEQ
    ids_p = jnp.concatenate([ids, jnp.zeros((pad,), jnp.int32)])
    pids_p = jnp.concatenate([pids, jnp.zeros((pad,), jnp.int32)])
    out = _emb_kernel(ids_p, pids_p, token_w, position_w)
    return out[None, :SEQ, :]
